# baseline (speedup 1.0000x reference)
import functools

import jax
import jax.numpy as jnp
from jax.experimental import pallas as pl
from jax.experimental.pallas import tpu as pltpu

LEAK = 0.6
BIAS = 1.6

BI = 1024
NBUF = 3


def _esn_kernel(w_ref, r_ref, u_ref, o_ref, wbuf, sem, *, res_dim, n):
    def start_copy(blk, slot):
        pltpu.make_async_copy(
            w_ref.at[pl.ds(blk * BI, BI), :],
            wbuf.at[slot],
            sem.at[slot],
        ).start()

    for b in range(NBUF - 1):
        start_copy(b, b)

    def body(i, carry):
        slot = jax.lax.rem(i, NBUF)
        pltpu.make_async_copy(
            w_ref.at[pl.ds(i * BI, BI), :], wbuf.at[slot], sem.at[slot],
        ).wait()
        col = i * BI
        c = col // res_dim
        rr = r_ref[:, pl.ds(c * res_dim, res_dim)]
        wt = wbuf[slot]
        pre = jax.lax.dot_general(
            rr, wt,
            dimension_numbers=(((1,), (1,)), ((), ())),
            preferred_element_type=jnp.float32,
        )
        pre = pre + u_ref[:, pl.ds(col, BI)] + BIAS
        r_slice = r_ref[:, pl.ds(col, BI)]
        o_ref[:, pl.ds(col, BI)] = LEAK * jnp.tanh(pre) + (1.0 - LEAK) * r_slice

        nxt = i + NBUF - 1

        @pl.when(nxt < n)
        def _():
            start_copy(nxt, jax.lax.rem(nxt, NBUF))

        return carry

    jax.lax.fori_loop(0, n, body, 0)


def kernel(proj_vars, res_state, wr):
    seq, chunks, res_dim = proj_vars.shape
    flat = chunks * res_dim
    u = proj_vars.reshape(seq, flat)
    r = res_state.reshape(seq, flat)
    w = wr.reshape(flat, res_dim)
    n = flat // BI

    body = functools.partial(_esn_kernel, res_dim=res_dim, n=n)

    out = pl.pallas_call(
        body,
        grid=(),
        in_specs=[
            pl.BlockSpec(memory_space=pltpu.MemorySpace.HBM),
            pl.BlockSpec((seq, flat), lambda: (0, 0)),
            pl.BlockSpec((seq, flat), lambda: (0, 0)),
        ],
        out_specs=pl.BlockSpec((seq, flat), lambda: (0, 0)),
        out_shape=jax.ShapeDtypeStruct((seq, flat), jnp.float32),
        scratch_shapes=[
            pltpu.VMEM((NBUF, BI, res_dim), jnp.float32),
            pltpu.SemaphoreType.DMA((NBUF,)),
        ],
    )(w, r, u)
    return out.reshape(seq, chunks, res_dim)


# speedup vs baseline: 1.0116x; 1.0116x over previous
import functools

import jax
import jax.numpy as jnp
from jax.experimental import pallas as pl
from jax.experimental.pallas import tpu as pltpu

LEAK = 0.6
BIAS = 1.6

BI = 512
NBUF = 4


def _esn_kernel(w_ref, r_ref, u_ref, o_ref, wbuf, sem, *, res_dim, n):
    def start_copy(blk, slot):
        pltpu.make_async_copy(
            w_ref.at[pl.ds(blk * BI, BI), :],
            wbuf.at[slot],
            sem.at[slot],
        ).start()

    for b in range(NBUF - 1):
        start_copy(b, b)

    def body(i, carry):
        slot = jax.lax.rem(i, NBUF)
        pltpu.make_async_copy(
            w_ref.at[pl.ds(i * BI, BI), :], wbuf.at[slot], sem.at[slot],
        ).wait()
        col = i * BI
        c = col // res_dim
        rr = r_ref[:, pl.ds(c * res_dim, res_dim)]
        wt = wbuf[slot]
        pre = jax.lax.dot_general(
            rr, wt,
            dimension_numbers=(((1,), (1,)), ((), ())),
            preferred_element_type=jnp.float32,
        )
        pre = pre + u_ref[:, pl.ds(col, BI)] + BIAS
        r_slice = r_ref[:, pl.ds(col, BI)]
        o_ref[:, pl.ds(col, BI)] = LEAK * jnp.tanh(pre) + (1.0 - LEAK) * r_slice

        nxt = i + NBUF - 1

        @pl.when(nxt < n)
        def _():
            start_copy(nxt, jax.lax.rem(nxt, NBUF))

        return carry

    jax.lax.fori_loop(0, n, body, 0)


def kernel(proj_vars, res_state, wr):
    seq, chunks, res_dim = proj_vars.shape
    flat = chunks * res_dim
    u = proj_vars.reshape(seq, flat)
    r = res_state.reshape(seq, flat)
    w = wr.reshape(flat, res_dim)
    n = flat // BI

    body = functools.partial(_esn_kernel, res_dim=res_dim, n=n)

    out = pl.pallas_call(
        body,
        grid=(),
        in_specs=[
            pl.BlockSpec(memory_space=pltpu.MemorySpace.HBM),
            pl.BlockSpec((seq, flat), lambda: (0, 0)),
            pl.BlockSpec((seq, flat), lambda: (0, 0)),
        ],
        out_specs=pl.BlockSpec((seq, flat), lambda: (0, 0)),
        out_shape=jax.ShapeDtypeStruct((seq, flat), jnp.float32),
        scratch_shapes=[
            pltpu.VMEM((NBUF, BI, res_dim), jnp.float32),
            pltpu.SemaphoreType.DMA((NBUF,)),
        ],
    )(w, r, u)
    return out.reshape(seq, chunks, res_dim)


# speedup vs baseline: 1.0154x; 1.0038x over previous
import jax
import jax.numpy as jnp
from jax.experimental import pallas as pl
from jax.experimental.pallas import tpu as pltpu

LEAK = 0.6
BIAS = 1.6

BI = 1024


def _esn_block(wr_ref, r_ref, u_ref, o_ref):
    i = pl.program_id(1)
    wt = wr_ref[...]
    rr = r_ref[...]
    pre = jax.lax.dot_general(
        rr, wt,
        dimension_numbers=(((1,), (1,)), ((), ())),
        preferred_element_type=jnp.float32,
    )
    pre = pre + u_ref[...] + BIAS
    r_slice = r_ref[:, pl.ds(i * BI, BI)]
    o_ref[...] = LEAK * jnp.tanh(pre) + (1.0 - LEAK) * r_slice


def kernel(proj_vars, res_state, wr):
    seq, chunks, res_dim = proj_vars.shape
    u = proj_vars.reshape(seq, chunks * res_dim)
    r = res_state.reshape(seq, chunks * res_dim)
    w = wr.reshape(chunks * res_dim, res_dim)
    n_i = res_dim // BI

    out = pl.pallas_call(
        _esn_block,
        grid=(chunks, n_i),
        in_specs=[
            pl.BlockSpec((BI, res_dim), lambda c, i: (c * (res_dim // BI) + i, 0)),
            pl.BlockSpec((seq, res_dim), lambda c, i: (0, c)),
            pl.BlockSpec((seq, BI), lambda c, i: (0, c * (res_dim // BI) + i)),
        ],
        out_specs=pl.BlockSpec((seq, BI), lambda c, i: (0, c * (res_dim // BI) + i)),
        out_shape=jax.ShapeDtypeStruct((seq, chunks * res_dim), jnp.float32),
        compiler_params=pltpu.CompilerParams(
            dimension_semantics=("parallel", "arbitrary"),
        ),
    )(w, r, u)
    return out.reshape(seq, chunks, res_dim)
